# Initial kernel scaffold; baseline (speedup 1.0000x reference)
import jax
import jax.numpy as jnp
from jax.experimental import pallas as pl


def kernel(feat_map, locations, offset, loc_pred):
    raise NotImplementedError("write your pallas kernel here")


# speedup vs baseline: 16.2211x; 16.2211x over previous
import functools

import jax
import jax.numpy as jnp
from jax import lax
from jax.experimental import pallas as pl
from jax.experimental.pallas import tpu as pltpu
from jax.experimental.pallas import tpu_sc as plsc

_B, _C, _H, _W = 2, 256, 128, 128
_P = 9
_N = _H * _W
_G = _B * _N * _P
_NW = 32
_PTS = _G // _NW
_CHUNK = 64
_ITERS = _PTS // _CHUNK


def _floor_clip(x, hi):
    t0 = x.astype(jnp.int32).astype(jnp.float32)
    fl = jnp.where(t0 > x, t0 - 1.0, t0)
    fl = jnp.minimum(jnp.maximum(fl, 0.0), hi)
    high = jnp.where(fl >= hi, fl, fl + 1.0)
    xe = jnp.where(fl >= hi, fl, x)
    frac = xe - fl
    return fl.astype(jnp.int32), high.astype(jnp.int32), frac, 1.0 - frac


def _sc_deform_body(table, ox, oy, out, ox_v, oy_v, idx_a, idx_b, w_a, w_b,
                    rows_a, rows_b, out_a, out_b, sem_ga, sem_gb, sem_oa,
                    sem_ob):
    wid = lax.axis_index("s") * 2 + lax.axis_index("c")
    g_base = wid * _PTS
    pltpu.sync_copy(ox.at[pl.ds(g_base, _PTS)], ox_v)
    pltpu.sync_copy(oy.at[pl.ds(g_base, _PTS)], oy_v)
    lanes = lax.iota(jnp.int32, 16)
    four = lanes * 4

    def stage_chunk(t, idx_v, w_v):
        for sub in range(_CHUNK // 16):
            gi = g_base + t * _CHUNK + sub * 16 + lanes
            b = gi // (_N * _P)
            n = (gi - b * (_N * _P)) // _P
            nx = lax.bitwise_and(n, _W - 1)
            ny = lax.shift_right_logical(n, 7)
            oxl = ox_v[pl.ds(t * _CHUNK + sub * 16, 16)]
            oyl = oy_v[pl.ds(t * _CHUNK + sub * 16, 16)]
            cw = ((nx.astype(jnp.float32) + 0.5) * 8.0
                  + oxl * 64.0 * 0.1) * 0.125
            ch = ((ny.astype(jnp.float32) + 0.5) * 8.0
                  + oyl * 64.0 * 0.1) * 0.125
            wl, wh, lw, hw_ = _floor_clip(cw, float(_W - 1))
            hl, hh, lh, hc = _floor_clip(ch, float(_H - 1))
            base_b = b * _N
            rl = base_b + hl * _W
            rh = base_b + hh * _W
            four_s = four + sub * 64
            plsc.store_scatter(idx_v, [four_s], rl + wl)
            plsc.store_scatter(idx_v, [four_s + 1], rl + wh)
            plsc.store_scatter(idx_v, [four_s + 2], rh + wl)
            plsc.store_scatter(idx_v, [four_s + 3], rh + wh)
            w_v[pl.ds(sub * 16, 16)] = hc * hw_
            w_v[pl.ds(_CHUNK + sub * 16, 16)] = hc * lw
            w_v[pl.ds(2 * _CHUNK + sub * 16, 16)] = lh * hw_
            w_v[pl.ds(3 * _CHUNK + sub * 16, 16)] = lh * lw

    def fire_gather(idx_v, rows_v, sem):
        return pltpu.async_copy(table.at[idx_v], rows_v, sem)

    def blend_chunk(w_v, rows_v, out_v):
        ilv = plsc.PackFormat.INTERLEAVED

        def blend(i, c):
            i16 = jnp.full((16,), i, dtype=jnp.int32)
            w1 = plsc.load_gather(w_v, [i16])
            w2 = plsc.load_gather(w_v, [i16 + _CHUNK])
            w3 = plsc.load_gather(w_v, [i16 + 2 * _CHUNK])
            w4 = plsc.load_gather(w_v, [i16 + 3 * _CHUNK])
            base = i * 4
            bc = lambda v: plsc.bitcast(v, jnp.bfloat16)
            for j in range(_C // 32):
                s = pl.ds(j * 16, 16)
                a0, a1 = plsc.unpack(bc(rows_v[base, s]), format=ilv)
                b0, b1 = plsc.unpack(bc(rows_v[base + 1, s]), format=ilv)
                c0, c1 = plsc.unpack(bc(rows_v[base + 2, s]), format=ilv)
                d0, d1 = plsc.unpack(bc(rows_v[base + 3, s]), format=ilv)
                r = 2 * i + j // 4
                col = (j % 4) * 32
                out_v[r, pl.ds(col, 16)] = (
                    w1 * a0 + w2 * b0 + w3 * c0 + w4 * d0)
                out_v[r, pl.ds(col + 16, 16)] = (
                    w1 * a1 + w2 * b1 + w3 * c1 + w4 * d1)
            return c

        lax.fori_loop(0, _CHUNK, blend, 0)

    def fire_out(out_v, t, sem):
        return pltpu.async_copy(
            out_v, out.at[pl.ds(2 * (g_base + t * _CHUNK), 2 * _CHUNK)], sem)

    stage_chunk(0, idx_a, w_a)
    fire_gather(idx_a, rows_a, sem_ga)

    def step(t, carry):
        ea = 2 * t
        ob = 2 * t + 1
        stage_chunk(ob, idx_b, w_b)
        fire_gather(idx_b, rows_b, sem_gb)
        pltpu.make_async_copy(table.at[idx_a], rows_a, sem_ga).wait()

        @pl.when(t > 0)
        def _():
            pltpu.make_async_copy(out_a, out.at[pl.ds(0, 2 * _CHUNK)],
                                  sem_oa).wait()

        blend_chunk(w_a, rows_a, out_a)
        fire_out(out_a, ea, sem_oa)

        @pl.when(t + 1 < _ITERS // 2)
        def _():
            stage_chunk(2 * t + 2, idx_a, w_a)
            fire_gather(idx_a, rows_a, sem_ga)

        pltpu.make_async_copy(table.at[idx_b], rows_b, sem_gb).wait()

        @pl.when(t > 0)
        def _():
            pltpu.make_async_copy(out_b, out.at[pl.ds(0, 2 * _CHUNK)],
                                  sem_ob).wait()

        blend_chunk(w_b, rows_b, out_b)
        fire_out(out_b, ob, sem_ob)
        return carry

    lax.fori_loop(0, _ITERS // 2, step, 0)
    pltpu.make_async_copy(out_a, out.at[pl.ds(0, 2 * _CHUNK)], sem_oa).wait()
    pltpu.make_async_copy(out_b, out.at[pl.ds(0, 2 * _CHUNK)], sem_ob).wait()


def _sc_deform(table, ox, oy):
    mesh = plsc.VectorSubcoreMesh(core_axis_name="c", subcore_axis_name="s")
    return pl.kernel(
        _sc_deform_body,
        out_type=jax.ShapeDtypeStruct((2 * _G, _C // 2), jnp.float32),
        mesh=mesh,
        scratch_types=[
            pltpu.VMEM((_PTS,), jnp.float32),
            pltpu.VMEM((_PTS,), jnp.float32),
            pltpu.VMEM((4 * _CHUNK,), jnp.int32),
            pltpu.VMEM((4 * _CHUNK,), jnp.int32),
            pltpu.VMEM((4 * _CHUNK,), jnp.float32),
            pltpu.VMEM((4 * _CHUNK,), jnp.float32),
            pltpu.VMEM((4 * _CHUNK, _C // 2), jnp.int32),
            pltpu.VMEM((4 * _CHUNK, _C // 2), jnp.int32),
            pltpu.VMEM((2 * _CHUNK, _C // 2), jnp.float32),
            pltpu.VMEM((2 * _CHUNK, _C // 2), jnp.float32),
            pltpu.SemaphoreType.DMA,
            pltpu.SemaphoreType.DMA,
            pltpu.SemaphoreType.DMA,
            pltpu.SemaphoreType.DMA,
        ],
        compiler_params=pltpu.CompilerParams(
            needs_layout_passes=False, use_tc_tiling_on_sc=True),
    )(table, ox, oy)


_BLK = 4096


def _tc_boxes_body(off_ref, lp_ref, out_ref):
    pid = pl.program_id(0)
    col = pid * _BLK + lax.broadcasted_iota(jnp.int32, (1, _BLK), 1)
    nx = lax.bitwise_and(col, _W - 1)
    ny = lax.bitwise_and(lax.shift_right_logical(col, 7), _H - 1)
    cx = (nx.astype(jnp.float32) + 0.5) * 8.0
    cy = (ny.astype(jnp.float32) + 0.5) * 8.0
    spx = (cx + off_ref[0:_P, :] * 64.0 * 0.1) + lp_ref[0:_P, :] * 64.0 * 0.5
    spy = ((cy + off_ref[_P:2 * _P, :] * 64.0 * 0.1)
           + lp_ref[_P:2 * _P, :] * 64.0 * 0.5)
    xmin = spx[0:1, :]
    ymin = spy[0:1, :]
    xmax = spx[0:1, :]
    ymax = spy[0:1, :]
    for p in range(1, _P):
        xmin = jnp.minimum(xmin, spx[p:p + 1, :])
        ymin = jnp.minimum(ymin, spy[p:p + 1, :])
        xmax = jnp.maximum(xmax, spx[p:p + 1, :])
        ymax = jnp.maximum(ymax, spy[p:p + 1, :])
    out_ref[...] = jnp.concatenate([xmin, ymin, xmax, ymax], axis=0)


def _tc_boxes(off_t, lp_t):
    bn = _B * _N
    spec = pl.BlockSpec((2 * _P, _BLK), lambda i: (0, i))
    return pl.pallas_call(
        _tc_boxes_body,
        grid=(bn // _BLK,),
        in_specs=[spec, spec],
        out_specs=pl.BlockSpec((4, _BLK), lambda i: (0, i)),
        out_shape=jax.ShapeDtypeStruct((4, bn), jnp.float32),
    )(off_t, lp_t)


@jax.jit
def kernel(feat_map, locations, offset, loc_pred):
    table = lax.bitcast_convert_type(
        feat_map.transpose(0, 2, 3, 1)
        .reshape(_B * _N, _C // 32, 2, 16)
        .transpose(0, 1, 3, 2)
        .reshape(_B * _N, _C // 2, 2)
        .astype(jnp.bfloat16),
        jnp.int32)
    ox = offset[..., 0].reshape(_G)
    oy = offset[..., 1].reshape(_G)
    deform = _sc_deform(table, ox, oy).reshape(_B, _N, _P, _C)
    bn = _B * _N
    off_t = jnp.transpose(offset.reshape(bn, _P, 2), (2, 1, 0)).reshape(
        2 * _P, bn)
    lp_t = jnp.transpose(loc_pred.reshape(bn, _P, 2), (2, 1, 0)).reshape(
        2 * _P, bn)
    boxes = _tc_boxes(off_t, lp_t).T.reshape(_B, _N, 4)
    return (deform, boxes)
